# baseline (speedup 1.0000x reference)
import functools

import jax
import jax.numpy as jnp
from jax import lax
from jax.experimental import pallas as pl
from jax.experimental.pallas import tpu as pltpu
from jax.experimental.pallas import tpu_sc as plsc

_NC = 2
_NS = 16
_NW = _NC * _NS
_L = 16

_MAGIC = 12582912.0
_K = 4


@functools.lru_cache(maxsize=None)
def _make_gather(n, dims):
    d0, d1, d2, d3 = dims
    per_w = n // _NW
    chunk = 4096
    n_chunks = per_w // chunk
    nb = chunk // 128

    s0, s1, s2, s3 = float(d0), float(d1), float(d2), float(d3)
    st0, st2, st3 = float(d1 * d2 * d3), float(d1 * d3), float(d1)

    mesh = plsc.VectorSubcoreMesh(core_axis_name="c", subcore_axis_name="s")

    @functools.partial(
        pl.kernel,
        out_type=jax.ShapeDtypeStruct((n,), jnp.float32),
        mesh=mesh,
        compiler_params=pltpu.CompilerParams(needs_layout_passes=False),
        scratch_types=[
            pltpu.VMEM((chunk * 4,), jnp.float32),
            pltpu.VMEM((chunk * 4,), jnp.float32),
            *[pltpu.VMEM((chunk // _K,), jnp.int32) for _ in range(2 * _K)],
            pltpu.VMEM((chunk,), jnp.float32),
            pltpu.VMEM((chunk,), jnp.float32),
            pltpu.SemaphoreType.DMA,
            pltpu.SemaphoreType.DMA,
            pltpu.SemaphoreType.DMA,
            pltpu.SemaphoreType.DMA,
            pltpu.SemaphoreType.DMA,
            pltpu.SemaphoreType.DMA,
        ],
    )
    def k(xs_hbm, table_hbm, out_hbm, *scratch):
        xs_v = scratch[0:2]
        idx_v = [scratch[2 : 2 + _K], scratch[2 + _K : 2 + 2 * _K]]
        val_v = scratch[2 + 2 * _K : 4 + 2 * _K]
        sems = scratch[4 + 2 * _K :]
        sem_x, sem_g, sem_o = sems[0:2], sems[2:4], sems[4:6]
        wid = lax.axis_index("s") * _NC + lax.axis_index("c")
        base0 = pl.multiple_of(wid * per_w, chunk)

        def load_xs(c, p):
            return pltpu.async_copy(
                xs_hbm.at[pl.ds((base0 + c * chunk) * 4, chunk * 4)],
                xs_v[p],
                sem_x[p],
            )

        def compute_idx(p):
            for q in range(_K):
                qbase = q * (chunk // _K)

                def blk_body(t, carry):
                    for j in range(128 // _L):
                        o = (qbase // 128 + t) * 512 + j * _L
                        fx = xs_v[p][pl.ds(o, _L)]
                        fy = xs_v[p][pl.ds(o + 128, _L)]
                        fz = xs_v[p][pl.ds(o + 256, _L)]
                        fw = xs_v[p][pl.ds(o + 384, _L)]
                        cx = jnp.clip((fx * s0 + _MAGIC) - _MAGIC, 0.0, s0 - 1.0)
                        cy = jnp.clip((fy * s1 + _MAGIC) - _MAGIC, 0.0, s1 - 1.0)
                        cz = jnp.clip((fz * s2 + _MAGIC) - _MAGIC, 0.0, s2 - 1.0)
                        cw = jnp.clip((fw * s3 + _MAGIC) - _MAGIC, 0.0, s3 - 1.0)
                        lin = cx * st0 + cz * st2 + cw * st3 + cy
                        idx_v[p][q][pl.ds(t * 128 + j * _L, _L)] = lin.astype(
                            jnp.int32
                        )
                    return carry

                lax.fori_loop(0, (chunk // _K) // 128, blk_body, 0)

        def fire_gather(p):
            cps = [
                pltpu.async_copy(
                    table_hbm.at[idx_v[p][q]],
                    val_v[p].at[pl.ds(q * (chunk // _K), chunk // _K)],
                    sem_g[p],
                )
                for q in range(_K)
            ]
            return cps

        def fire_out(c, p):
            return pltpu.async_copy(
                val_v[p], out_hbm.at[pl.ds(base0 + c * chunk, chunk)], sem_o[p]
            )

        cp_x = [load_xs(0, 0), load_xs(1, 1)]
        cp_g = [None, None]
        cp_o = [None, None]
        for c in range(n_chunks):
            p = c & 1
            cp_x[p].wait()
            compute_idx(p)
            if c + 2 < n_chunks:
                cp_x[p] = load_xs(c + 2, p)
            if cp_o[p] is not None:
                cp_o[p].wait()
            cp_g[p] = fire_gather(p)
            if c >= 1:
                for cp in cp_g[1 - p]:
                    cp.wait()
                cp_o[1 - p] = fire_out(c - 1, 1 - p)
        last = (n_chunks - 1) & 1
        for cp in cp_g[last]:
            cp.wait()
        cp_o[last] = fire_out(n_chunks - 1, last)
        cp_o[1 - last].wait()
        cp_o[last].wait()

    return k


def kernel(xs, data):
    n = xs.shape[0]
    dims = data.shape[:4]
    xs_flat = xs.reshape(n // 128, 128, 4).transpose(0, 2, 1).reshape(-1)
    table = data.transpose(0, 2, 3, 4, 1).reshape(-1)
    out = _make_gather(n, dims)(xs_flat, table)
    return out.reshape(n, 1)


# speedup vs baseline: 1.1780x; 1.1780x over previous
import functools

import jax
import jax.numpy as jnp
from jax import lax
from jax.experimental import pallas as pl
from jax.experimental.pallas import tpu as pltpu
from jax.experimental.pallas import tpu_sc as plsc

_NC = 2
_NS = 16
_NW = _NC * _NS
_L = 16

_MAGIC = 12582912.0


@functools.lru_cache(maxsize=None)
def _make_gather(n, dims):
    d0, d1, d2, d3 = dims
    per_w = n // _NW
    chunk = 4096
    n_chunks = per_w // chunk
    nb = chunk // 128

    s0, s1, s2, s3 = float(d0), float(d1), float(d2), float(d3)
    st0, st2, st3 = float(d1 * d2 * d3), float(d1 * d3), float(d1)

    mesh = plsc.VectorSubcoreMesh(core_axis_name="c", subcore_axis_name="s")

    @functools.partial(
        pl.kernel,
        out_type=jax.ShapeDtypeStruct((n,), jnp.float32),
        mesh=mesh,
        compiler_params=pltpu.CompilerParams(needs_layout_passes=False),
        scratch_types=[
            pltpu.VMEM((chunk * 4,), jnp.float32),
            pltpu.VMEM((chunk * 4,), jnp.float32),
            pltpu.VMEM((chunk,), jnp.float32),
            pltpu.VMEM((chunk,), jnp.float32),
            pltpu.SemaphoreType.DMA,
            pltpu.SemaphoreType.DMA,
            pltpu.SemaphoreType.DMA,
            pltpu.SemaphoreType.DMA,
            pltpu.SemaphoreType.DMA,
            pltpu.SemaphoreType.DMA,
        ],
    )
    def k(xs_hbm, table_hbm, out_hbm, *scratch):
        xs_v, val_v = scratch[0:2], scratch[2:4]
        sem_x, sem_g, sem_o = scratch[4:6], scratch[6:8], scratch[8:10]
        wid = lax.axis_index("s") * _NC + lax.axis_index("c")
        base0 = pl.multiple_of(wid * per_w, chunk)

        def load_xs(c, p):
            return pltpu.async_copy(
                xs_hbm.at[pl.ds((base0 + c * chunk) * 4, chunk * 4)],
                xs_v[p],
                sem_x[p],
            )

        def compute_fire(p):
            def blk_body(t, carry):
                for j in range(128 // _L):
                    o = t * 512 + j * _L
                    fx = xs_v[p][pl.ds(o, _L)]
                    fy = xs_v[p][pl.ds(o + 128, _L)]
                    fz = xs_v[p][pl.ds(o + 256, _L)]
                    fw = xs_v[p][pl.ds(o + 384, _L)]
                    cx = jnp.clip((fx * s0 + _MAGIC) - _MAGIC, 0.0, s0 - 1.0)
                    cy = jnp.clip((fy * s1 + _MAGIC) - _MAGIC, 0.0, s1 - 1.0)
                    cz = jnp.clip((fz * s2 + _MAGIC) - _MAGIC, 0.0, s2 - 1.0)
                    cw = jnp.clip((fw * s3 + _MAGIC) - _MAGIC, 0.0, s3 - 1.0)
                    lin = (cx * st0 + cz * st2 + cw * st3 + cy).astype(jnp.int32)
                    pltpu.async_copy(
                        table_hbm.at[lin],
                        val_v[p].at[pl.ds(t * 128 + j * _L, _L)],
                        sem_g[p],
                    )
                return carry

            lax.fori_loop(0, nb, blk_body, 0)

        def drain_gathers(p):
            pltpu.make_async_copy(
                table_hbm.at[pl.ds(0, chunk)], val_v[p], sem_g[p]
            ).wait()

        def fire_out(c, p):
            return pltpu.async_copy(
                val_v[p], out_hbm.at[pl.ds(base0 + c * chunk, chunk)], sem_o[p]
            )

        cp_x = [load_xs(0, 0), load_xs(1, 1)]
        cp_o = [None, None]
        for c in range(n_chunks):
            p = c & 1
            cp_x[p].wait()
            if cp_o[p] is not None:
                cp_o[p].wait()
            compute_fire(p)
            if c + 2 < n_chunks:
                cp_x[p] = load_xs(c + 2, p)
            if c >= 1:
                drain_gathers(1 - p)
                cp_o[1 - p] = fire_out(c - 1, 1 - p)
        last = (n_chunks - 1) & 1
        drain_gathers(last)
        cp_o[last] = fire_out(n_chunks - 1, last)
        cp_o[1 - last].wait()
        cp_o[last].wait()

    return k


def kernel(xs, data):
    n = xs.shape[0]
    dims = data.shape[:4]
    xs_flat = xs.reshape(n // 128, 128, 4).transpose(0, 2, 1).reshape(-1)
    table = data.transpose(0, 2, 3, 4, 1).reshape(-1)
    out = _make_gather(n, dims)(xs_flat, table)
    return out.reshape(n, 1)


# speedup vs baseline: 1.2047x; 1.0227x over previous
import functools

import jax
import jax.numpy as jnp
from jax import lax
from jax.experimental import pallas as pl
from jax.experimental.pallas import tpu as pltpu
from jax.experimental.pallas import tpu_sc as plsc

_NC = 2
_NS = 16
_NW = _NC * _NS
_L = 16

_MAGIC = 12582912.0


@functools.lru_cache(maxsize=None)
def _make_gather(n, dims):
    d0, d1, d2, d3 = dims
    per_w = n // _NW
    chunk = 8192
    n_chunks = per_w // chunk
    nb = chunk // 128

    s0, s1, s2, s3 = float(d0), float(d1), float(d2), float(d3)
    st0, st2, st3 = float(d1 * d2 * d3), float(d1 * d3), float(d1)

    mesh = plsc.VectorSubcoreMesh(core_axis_name="c", subcore_axis_name="s")

    @functools.partial(
        pl.kernel,
        out_type=jax.ShapeDtypeStruct((n,), jnp.float32),
        mesh=mesh,
        compiler_params=pltpu.CompilerParams(needs_layout_passes=False),
        scratch_types=[
            pltpu.VMEM((chunk * 4,), jnp.float32),
            pltpu.VMEM((chunk * 4,), jnp.float32),
            pltpu.VMEM((chunk,), jnp.float32),
            pltpu.VMEM((chunk,), jnp.float32),
            pltpu.SemaphoreType.DMA,
            pltpu.SemaphoreType.DMA,
            pltpu.SemaphoreType.DMA,
            pltpu.SemaphoreType.DMA,
            pltpu.SemaphoreType.DMA,
            pltpu.SemaphoreType.DMA,
        ],
    )
    def k(xs_hbm, table_hbm, out_hbm, *scratch):
        xs_v, val_v = scratch[0:2], scratch[2:4]
        sem_x, sem_g, sem_o = scratch[4:6], scratch[6:8], scratch[8:10]
        wid = lax.axis_index("s") * _NC + lax.axis_index("c")
        base0 = pl.multiple_of(wid * per_w, chunk)

        def load_xs(c, p):
            return pltpu.async_copy(
                xs_hbm.at[pl.ds((base0 + c * chunk) * 4, chunk * 4)],
                xs_v[p],
                sem_x[p],
            )

        def compute_fire(p):
            def blk_body(t, carry):
                for j in range(128 // _L):
                    o = t * 512 + j * _L
                    fx = xs_v[p][pl.ds(o, _L)]
                    fy = xs_v[p][pl.ds(o + 128, _L)]
                    fz = xs_v[p][pl.ds(o + 256, _L)]
                    fw = xs_v[p][pl.ds(o + 384, _L)]
                    cx = jnp.clip((fx * s0 + _MAGIC) - _MAGIC, 0.0, s0 - 1.0)
                    cy = jnp.clip((fy * s1 + _MAGIC) - _MAGIC, 0.0, s1 - 1.0)
                    cz = jnp.clip((fz * s2 + _MAGIC) - _MAGIC, 0.0, s2 - 1.0)
                    cw = jnp.clip((fw * s3 + _MAGIC) - _MAGIC, 0.0, s3 - 1.0)
                    lin = (cx * st0 + cz * st2 + cw * st3 + cy).astype(jnp.int32)
                    pltpu.async_copy(
                        table_hbm.at[lin],
                        val_v[p].at[pl.ds(t * 128 + j * _L, _L)],
                        sem_g[p],
                    )
                return carry

            lax.fori_loop(0, nb, blk_body, 0)

        def drain_gathers(p):
            pltpu.make_async_copy(
                table_hbm.at[pl.ds(0, chunk)], val_v[p], sem_g[p]
            ).wait()

        def fire_out(c, p):
            return pltpu.async_copy(
                val_v[p], out_hbm.at[pl.ds(base0 + c * chunk, chunk)], sem_o[p]
            )

        cp_x = [load_xs(0, 0), load_xs(1, 1)]
        cp_o = [None, None]
        for c in range(n_chunks):
            p = c & 1
            cp_x[p].wait()
            if cp_o[p] is not None:
                cp_o[p].wait()
            compute_fire(p)
            if c + 2 < n_chunks:
                cp_x[p] = load_xs(c + 2, p)
            if c >= 1:
                drain_gathers(1 - p)
                cp_o[1 - p] = fire_out(c - 1, 1 - p)
        last = (n_chunks - 1) & 1
        drain_gathers(last)
        cp_o[last] = fire_out(n_chunks - 1, last)
        cp_o[1 - last].wait()
        cp_o[last].wait()

    return k


def kernel(xs, data):
    n = xs.shape[0]
    dims = data.shape[:4]
    xs_flat = xs.reshape(n // 128, 128, 4).transpose(0, 2, 1).reshape(-1)
    table = data.transpose(0, 2, 3, 4, 1).reshape(-1)
    out = _make_gather(n, dims)(xs_flat, table)
    return out.reshape(n, 1)


# speedup vs baseline: 1.2194x; 1.0121x over previous
import functools

import jax
import jax.numpy as jnp
from jax import lax
from jax.experimental import pallas as pl
from jax.experimental.pallas import tpu as pltpu
from jax.experimental.pallas import tpu_sc as plsc

_NC = 2
_NS = 16
_NW = _NC * _NS
_L = 16

_MAGIC = 12582912.0


@functools.lru_cache(maxsize=None)
def _make_gather(n, dims):
    d0, d1, d2, d3 = dims
    per_w = n // _NW
    chunk = 8192
    n_chunks = per_w // chunk
    nb = chunk // 128

    s0, s1, s2, s3 = float(d0), float(d1), float(d2), float(d3)
    st0, st2, st3 = float(d1 * d2 * d3), float(d1 * d3), float(d1)

    mesh = plsc.VectorSubcoreMesh(core_axis_name="c", subcore_axis_name="s")

    @functools.partial(
        pl.kernel,
        out_type=jax.ShapeDtypeStruct((n,), jnp.float32),
        mesh=mesh,
        compiler_params=pltpu.CompilerParams(needs_layout_passes=False),
        scratch_types=[
            pltpu.VMEM((chunk * 4,), jnp.float32),
            pltpu.VMEM((chunk * 4,), jnp.float32),
            pltpu.VMEM((chunk,), jnp.float32),
            pltpu.VMEM((chunk,), jnp.float32),
            pltpu.SemaphoreType.DMA,
            pltpu.SemaphoreType.DMA,
            pltpu.SemaphoreType.DMA,
            pltpu.SemaphoreType.DMA,
            pltpu.SemaphoreType.DMA,
            pltpu.SemaphoreType.DMA,
        ],
    )
    def k(xs_hbm, table_hbm, out_hbm, *scratch):
        xs_v, val_v = scratch[0:2], scratch[2:4]
        sem_x, sem_g, sem_o = scratch[4:6], scratch[6:8], scratch[8:10]
        wid = lax.axis_index("s") * _NC + lax.axis_index("c")
        base0 = pl.multiple_of(wid * per_w, chunk)

        def load_xs(c, p):
            return pltpu.async_copy(
                xs_hbm.at[pl.ds((base0 + c * chunk) * 4, chunk * 4)],
                xs_v[p],
                sem_x[p],
            )

        def compute_fire(p):
            def blk_body(it, carry):
                t = it >> 1
                j0 = (it & 1) * 4
                for jj in range(4):
                    j = j0 + jj
                    o = t * 512 + j * _L
                    fx = xs_v[p][pl.ds(o, _L)]
                    fy = xs_v[p][pl.ds(o + 128, _L)]
                    fz = xs_v[p][pl.ds(o + 256, _L)]
                    fw = xs_v[p][pl.ds(o + 384, _L)]
                    cx = jnp.clip((fx * s0 + _MAGIC) - _MAGIC, 0.0, s0 - 1.0)
                    cy = jnp.clip((fy * s1 + _MAGIC) - _MAGIC, 0.0, s1 - 1.0)
                    cz = jnp.clip((fz * s2 + _MAGIC) - _MAGIC, 0.0, s2 - 1.0)
                    cw = jnp.clip((fw * s3 + _MAGIC) - _MAGIC, 0.0, s3 - 1.0)
                    lin = (cx * st0 + cz * st2 + cw * st3 + cy).astype(jnp.int32)
                    pltpu.async_copy(
                        table_hbm.at[lin],
                        val_v[p].at[pl.ds(t * 128 + j * _L, _L)],
                        sem_g[p],
                    )
                return carry

            lax.fori_loop(0, 2 * nb, blk_body, 0)

        def drain_gathers(p):
            pltpu.make_async_copy(
                table_hbm.at[pl.ds(0, chunk)], val_v[p], sem_g[p]
            ).wait()

        def fire_out(c, p):
            return pltpu.async_copy(
                val_v[p], out_hbm.at[pl.ds(base0 + c * chunk, chunk)], sem_o[p]
            )

        cp_x = [load_xs(0, 0), load_xs(1, 1)]
        cp_o = [None, None]
        for c in range(n_chunks):
            p = c & 1
            cp_x[p].wait()
            if cp_o[p] is not None:
                cp_o[p].wait()
            compute_fire(p)
            if c + 2 < n_chunks:
                cp_x[p] = load_xs(c + 2, p)
            if c >= 1:
                drain_gathers(1 - p)
                cp_o[1 - p] = fire_out(c - 1, 1 - p)
        last = (n_chunks - 1) & 1
        drain_gathers(last)
        cp_o[last] = fire_out(n_chunks - 1, last)
        cp_o[1 - last].wait()
        cp_o[last].wait()

    return k


def kernel(xs, data):
    n = xs.shape[0]
    dims = data.shape[:4]
    xs_flat = xs.reshape(n // 128, 128, 4).transpose(0, 2, 1).reshape(-1)
    table = data.transpose(0, 2, 3, 4, 1).reshape(-1)
    out = _make_gather(n, dims)(xs_flat, table)
    return out.reshape(n, 1)
